# Initial kernel scaffold; baseline (speedup 1.0000x reference)
import jax
import jax.numpy as jnp
from jax.experimental import pallas as pl


def kernel(x, edge_index, Wc, bc, We, be):
    raise NotImplementedError("write your pallas kernel here")


# speedup vs baseline: 167.2243x; 167.2243x over previous
import functools

import jax
import jax.numpy as jnp
import numpy as np
from jax import lax
from jax.experimental import pallas as pl
from jax.experimental.pallas import tpu as pltpu
from jax.experimental.pallas import tpu_sc as plsc

_NUM_PARTITIONS = 4
_NUM_TRIALS = 5

_GATHER_DN = lax.GatherDimensionNumbers(
    offset_dims=(), collapsed_slice_dims=(0,), start_index_map=(0,)
)


def _allreduce16(v, lanes):
    for sh in (1, 2, 4, 8):
        idx = (lanes ^ sh).reshape(16, 1)
        v = v + lax.gather(
            v, idx, _GATHER_DN, (1,),
            mode=lax.GatherScatterMode.PROMISE_IN_BOUNDS,
        )
    return v

_U32 = np.uint32


def _tf2x32(k1, k2, x1, x2):
    rot0 = (13, 15, 26, 6)
    rot1 = (17, 29, 16, 24)
    ks0 = _U32(k1)
    ks1 = _U32(k2)
    ks2 = _U32(ks0 ^ ks1 ^ _U32(0x1BD11BDA))
    a = (x1 + ks0).astype(_U32)
    b = (x2 + ks1).astype(_U32)

    def rnd(a, b, r):
        a = (a + b).astype(_U32)
        b = ((b << _U32(r)) | (b >> _U32(32 - r))).astype(_U32)
        return a, (a ^ b).astype(_U32)

    for r in rot0:
        a, b = rnd(a, b, r)
    a = (a + ks1).astype(_U32); b = (b + ks2 + _U32(1)).astype(_U32)
    for r in rot1:
        a, b = rnd(a, b, r)
    a = (a + ks2).astype(_U32); b = (b + ks0 + _U32(2)).astype(_U32)
    for r in rot0:
        a, b = rnd(a, b, r)
    a = (a + ks0).astype(_U32); b = (b + ks1 + _U32(3)).astype(_U32)
    for r in rot1:
        a, b = rnd(a, b, r)
    a = (a + ks1).astype(_U32); b = (b + ks2 + _U32(4)).astype(_U32)
    for r in rot0:
        a, b = rnd(a, b, r)
    a = (a + ks2).astype(_U32); b = (b + ks0 + _U32(5)).astype(_U32)
    return a, b


def _np_permutation(key, n):
    x = np.arange(n)
    num_rounds = int(
        np.ceil(3 * np.log(max(1, n)) / np.log(2**32 - 1))
    )
    for _ in range(num_rounds):
        a, b = _tf2x32(
            key[0], key[1], np.array([0, 0], _U32), np.array([0, 1], _U32)
        )
        key, subkey = (a[0], b[0]), (a[1], b[1])
        ra, rb = _tf2x32(
            subkey[0], subkey[1],
            np.zeros(n, _U32), np.arange(n, dtype=_U32),
        )
        order = np.argsort((ra ^ rb).astype(_U32), kind="stable")
        x = x[order]
    return x


_CODES_CACHE = {}


def _partition_codes(n: int) -> np.ndarray:
    if n not in _CODES_CACHE:
        old_err = np.seterr(over="ignore")
        try:
            base = (_U32(0), _U32(42))
            psize = n // _NUM_PARTITIONS
            part_of_pos = np.minimum(
                np.arange(n) // psize, _NUM_PARTITIONS - 1
            ).astype(np.int32)
            code = np.zeros((n,), np.int32)
            for t in range(_NUM_TRIALS):
                ka, kb = _tf2x32(
                    base[0], base[1], np.array([0], _U32),
                    np.array([t], _U32),
                )
                perm = _np_permutation((ka[0], kb[0]), n)
                part = np.zeros((n,), np.int32)
                part[perm] = part_of_pos
                code |= part << (2 * t)
        finally:
            np.seterr(**old_err)
        _CODES_CACHE[n] = code
    return _CODES_CACHE[n]


def _dense_body(x_ref, wct_ref, bc_ref, wet_ref, be_ref,
                chat_ref, fhat_ref, nc_ref, nf_ref):
    x = x_ref[...]
    dn = (((1,), (0,)), ((), ()))
    c = jnp.tanh(
        lax.dot_general(x, wct_ref[...], dn, preferred_element_type=jnp.float32)
        + bc_ref[...]
    )
    f = jnp.tanh(
        lax.dot_general(x, wet_ref[...], dn, preferred_element_type=jnp.float32)
        + be_ref[...]
    )
    nc = jnp.sqrt(jnp.sum(c * c, axis=1, keepdims=True))
    nf = jnp.sqrt(jnp.sum(f * f, axis=1, keepdims=True))
    chat_ref[...] = c * jnp.where(nc > 0, 1.0 / nc, 0.0)
    fhat_ref[...] = f * jnp.where(nf > 0, 1.0 / nf, 0.0)
    nc_ref[...] = nc
    nf_ref[...] = nf


def _make_sc_kernel(n: int, d: int, e_per: int, ch: int, nw: int, nc_cores: int):
    mesh = plsc.VectorSubcoreMesh(core_axis_name="c", subcore_axis_name="s")
    n_chunks = e_per // ch
    n_groups = ch // 16
    nbuckets = _NUM_TRIALS * _NUM_PARTITIONS
    acc_rows = 2 * nbuckets + 1

    @functools.partial(
        pl.kernel,
        mesh=mesh,
        out_type=jax.ShapeDtypeStruct((nw, acc_rows, 16), jnp.float32),
        scratch_types=[
            pltpu.VMEM((ch,), jnp.int32),
            pltpu.VMEM((ch,), jnp.int32),
            pltpu.VMEM((ch, d), jnp.float32),
            pltpu.VMEM((ch, d), jnp.float32),
            pltpu.VMEM((n,), jnp.float32),
            pltpu.VMEM((n,), jnp.float32),
            pltpu.VMEM((n,), jnp.int32),
            pltpu.VMEM((acc_rows, 16), jnp.float32),
            pltpu.SemaphoreType.DMA,
            pltpu.SemaphoreType.DMA,
        ],
        compiler_params=pltpu.CompilerParams(needs_layout_passes=False),
    )
    def sc_kernel(chat_hbm, fhat_hbm, nc_hbm, nf_hbm, codes_hbm, src_hbm,
                  dst_hbm, out_hbm, src_v, dst_v, crows, frows, nc_v, nf_v,
                  codes_v, acc_v, sem_c, sem_f):
        wid = lax.axis_index("s") * nc_cores + lax.axis_index("c")
        base = wid * e_per

        pltpu.sync_copy(nc_hbm, nc_v)
        pltpu.sync_copy(nf_hbm, nf_v)
        pltpu.sync_copy(codes_hbm, codes_v)

        zeros16 = jnp.zeros((16,), jnp.float32)
        for b in range(acc_rows):
            acc_v[b, :] = zeros16

        lanes = lax.iota(jnp.int32, 16)
        ones16 = jnp.ones((16,), jnp.float32)

        def chunk_body(cidx, _):
            off = base + cidx * ch
            pltpu.sync_copy(src_hbm.at[pl.ds(off, ch)], src_v)
            pltpu.sync_copy(dst_hbm.at[pl.ds(off, ch)], dst_v)
            cp_c = pltpu.async_copy(chat_hbm.at[src_v], crows, sem_c)
            cp_f = pltpu.async_copy(fhat_hbm.at[dst_v], frows, sem_f)
            cp_c.wait()
            cp_f.wait()

            def group_body(g, _):
                g16 = g * 16
                s16 = src_v[pl.ds(g16, 16)]
                d16 = dst_v[pl.ds(g16, 16)]
                acc = jnp.zeros((16,), jnp.float32)
                for j in range(16):
                    e = g16 + j
                    parts = [
                        crows[e, pl.ds(k * 16, 16)] * frows[e, pl.ds(k * 16, 16)]
                        for k in range(d // 16)
                    ]
                    while len(parts) > 1:
                        parts = [
                            parts[i] + parts[i + 1]
                            for i in range(0, len(parts) - 1, 2)
                        ] + ([parts[-1]] if len(parts) % 2 else [])
                    red = _allreduce16(parts[0], lanes)
                    acc = jnp.where(lanes == j, red, acc)
                ncv = plsc.load_gather(nc_v, [s16])
                nfv = plsc.load_gather(nf_v, [d16])
                csv = plsc.load_gather(codes_v, [s16])
                cdv = plsc.load_gather(codes_v, [d16])
                scale = jnp.minimum(1.0, ncv * nfv * 1e8)
                ce = acc * scale
                acc_v[nbuckets, :] = acc_v[nbuckets, :] + ce
                for t in range(_NUM_TRIALS):
                    ps = (csv >> (2 * t)) & 3
                    pd = (cdv >> (2 * t)) & 3
                    m = ps == pd
                    bidx = ps + (4 * t)
                    plsc.addupdate_scatter(acc_v, [bidx, lanes], ce, mask=m)
                    plsc.addupdate_scatter(
                        acc_v, [bidx + (nbuckets + 1), lanes], ones16, mask=m
                    )
                return 0

            lax.fori_loop(0, n_groups, group_body, 0)
            return 0

        lax.fori_loop(0, n_chunks, chunk_body, 0)
        pltpu.sync_copy(acc_v, out_hbm.at[wid])

    return sc_kernel


def kernel(x, edge_index, Wc, bc, We, be):
    n, d = x.shape
    e = edge_index.shape[1]
    info = plsc.get_sparse_core_info()
    nc_cores, ns = info.num_cores, info.num_subcores
    nw = nc_cores * ns
    assert e % nw == 0
    e_per = e // nw
    ch = 80
    assert e_per % ch == 0 and ch % 16 == 0

    dense = pl.pallas_call(
        _dense_body,
        out_shape=[
            jax.ShapeDtypeStruct((n, d), jnp.float32),
            jax.ShapeDtypeStruct((n, d), jnp.float32),
            jax.ShapeDtypeStruct((n, 1), jnp.float32),
            jax.ShapeDtypeStruct((n, 1), jnp.float32),
        ],
    )
    chat, fhat, ncn, nfn = dense(
        x,
        Wc.T,
        bc.reshape(1, d),
        We.T,
        be.reshape(1, d),
    )
    ncn = ncn.reshape(n)
    nfn = nfn.reshape(n)
    codes = jnp.asarray(_partition_codes(n))
    src = edge_index[0].astype(jnp.int32)
    dst = edge_index[1].astype(jnp.int32)

    sc = _make_sc_kernel(n, d, e_per, ch, nw, nc_cores)
    parts = sc(chat, fhat, ncn, nfn, codes, src, dst)

    tot = jnp.sum(parts, axis=(0, 2))
    nb = _NUM_TRIALS * _NUM_PARTITIONS
    sums = tot[0:nb].reshape(_NUM_TRIALS, _NUM_PARTITIONS)
    gsum = tot[nb]
    cnts = tot[nb + 1:2 * nb + 1].reshape(_NUM_TRIALS, _NUM_PARTITIONS)

    ei_whole = gsum / e
    pei = sums / jnp.maximum(cnts, 1.0)
    pvalid = cnts > 0
    nval = jnp.sum(pvalid, axis=1)
    avg = jnp.sum(jnp.where(pvalid, pei, 0.0), axis=1) / jnp.maximum(
        nval, 1
    ).astype(jnp.float32)
    tvalid = nval > 0
    any_valid = jnp.any(tvalid)
    min_pei_valid = jnp.min(jnp.where(tvalid, avg, jnp.inf))
    min_pei = jnp.where(any_valid, min_pei_valid, jnp.array(jnp.inf, jnp.float32))
    phi = jnp.where(
        any_valid,
        jnp.maximum(ei_whole - min_pei_valid, 0.0),
        jnp.maximum(ei_whole, 0.0),
    )
    return phi, ei_whole, min_pei


# speedup vs baseline: 392.3661x; 2.3463x over previous
import functools

import jax
import jax.numpy as jnp
import numpy as np
from jax import lax
from jax.experimental import pallas as pl
from jax.experimental.pallas import tpu as pltpu
from jax.experimental.pallas import tpu_sc as plsc

_NUM_PARTITIONS = 4
_NUM_TRIALS = 5

_GATHER_DN = lax.GatherDimensionNumbers(
    offset_dims=(), collapsed_slice_dims=(0,), start_index_map=(0,)
)


def _allreduce16(v, lanes):
    for sh in (1, 2, 4, 8):
        idx = (lanes ^ sh).reshape(16, 1)
        v = v + lax.gather(
            v, idx, _GATHER_DN, (1,),
            mode=lax.GatherScatterMode.PROMISE_IN_BOUNDS,
        )
    return v

_U32 = np.uint32


def _tf2x32(k1, k2, x1, x2):
    rot0 = (13, 15, 26, 6)
    rot1 = (17, 29, 16, 24)
    ks0 = _U32(k1)
    ks1 = _U32(k2)
    ks2 = _U32(ks0 ^ ks1 ^ _U32(0x1BD11BDA))
    a = (x1 + ks0).astype(_U32)
    b = (x2 + ks1).astype(_U32)

    def rnd(a, b, r):
        a = (a + b).astype(_U32)
        b = ((b << _U32(r)) | (b >> _U32(32 - r))).astype(_U32)
        return a, (a ^ b).astype(_U32)

    for r in rot0:
        a, b = rnd(a, b, r)
    a = (a + ks1).astype(_U32); b = (b + ks2 + _U32(1)).astype(_U32)
    for r in rot1:
        a, b = rnd(a, b, r)
    a = (a + ks2).astype(_U32); b = (b + ks0 + _U32(2)).astype(_U32)
    for r in rot0:
        a, b = rnd(a, b, r)
    a = (a + ks0).astype(_U32); b = (b + ks1 + _U32(3)).astype(_U32)
    for r in rot1:
        a, b = rnd(a, b, r)
    a = (a + ks1).astype(_U32); b = (b + ks2 + _U32(4)).astype(_U32)
    for r in rot0:
        a, b = rnd(a, b, r)
    a = (a + ks2).astype(_U32); b = (b + ks0 + _U32(5)).astype(_U32)
    return a, b


def _np_permutation(key, n):
    x = np.arange(n)
    num_rounds = int(
        np.ceil(3 * np.log(max(1, n)) / np.log(2**32 - 1))
    )
    for _ in range(num_rounds):
        a, b = _tf2x32(
            key[0], key[1], np.array([0, 0], _U32), np.array([0, 1], _U32)
        )
        key, subkey = (a[0], b[0]), (a[1], b[1])
        ra, rb = _tf2x32(
            subkey[0], subkey[1],
            np.zeros(n, _U32), np.arange(n, dtype=_U32),
        )
        order = np.argsort((ra ^ rb).astype(_U32), kind="stable")
        x = x[order]
    return x


_CODES_CACHE = {}


def _partition_codes(n: int) -> np.ndarray:
    if n not in _CODES_CACHE:
        old_err = np.seterr(over="ignore")
        try:
            base = (_U32(0), _U32(42))
            psize = n // _NUM_PARTITIONS
            part_of_pos = np.minimum(
                np.arange(n) // psize, _NUM_PARTITIONS - 1
            ).astype(np.int32)
            code = np.zeros((n,), np.int32)
            for t in range(_NUM_TRIALS):
                ka, kb = _tf2x32(
                    base[0], base[1], np.array([0], _U32),
                    np.array([t], _U32),
                )
                perm = _np_permutation((ka[0], kb[0]), n)
                part = np.zeros((n,), np.int32)
                part[perm] = part_of_pos
                code |= part << (2 * t)
        finally:
            np.seterr(**old_err)
        _CODES_CACHE[n] = code
    return _CODES_CACHE[n]


def _dense_body(x_ref, wct_ref, bc_ref, wet_ref, be_ref,
                chat_ref, fhat_ref, nc_ref, nf_ref):
    x = x_ref[...]
    dn = (((1,), (0,)), ((), ()))
    c = jnp.tanh(
        lax.dot_general(x, wct_ref[...], dn, preferred_element_type=jnp.float32)
        + bc_ref[...]
    )
    f = jnp.tanh(
        lax.dot_general(x, wet_ref[...], dn, preferred_element_type=jnp.float32)
        + be_ref[...]
    )
    nc = jnp.sqrt(jnp.sum(c * c, axis=1, keepdims=True))
    nf = jnp.sqrt(jnp.sum(f * f, axis=1, keepdims=True))
    chat_ref[...] = c * jnp.where(nc > 0, 1.0 / nc, 0.0)
    fhat_ref[...] = f * jnp.where(nf > 0, 1.0 / nf, 0.0)
    nc_ref[...] = nc
    nf_ref[...] = nf


def _make_sc_kernel(n: int, d: int, e_per: int, ch: int, nw: int, nc_cores: int):
    mesh = plsc.VectorSubcoreMesh(core_axis_name="c", subcore_axis_name="s")
    n_chunks = e_per // ch
    n_groups = ch // 16
    nbuckets = _NUM_TRIALS * _NUM_PARTITIONS
    acc_rows = 2 * nbuckets + 1

    assert n_chunks % 2 == 1, "pipeline epilogue below assumes odd n_chunks"

    @functools.partial(
        pl.kernel,
        mesh=mesh,
        out_type=jax.ShapeDtypeStruct((nw, acc_rows, 16), jnp.float32),
        scratch_types=[
            pltpu.VMEM((e_per,), jnp.int32),
            pltpu.VMEM((e_per,), jnp.int32),
            pltpu.VMEM((ch, d), jnp.float32),
            pltpu.VMEM((ch, d), jnp.float32),
            pltpu.VMEM((ch, d), jnp.float32),
            pltpu.VMEM((ch, d), jnp.float32),
            pltpu.VMEM((n,), jnp.float32),
            pltpu.VMEM((n,), jnp.float32),
            pltpu.VMEM((n,), jnp.int32),
            pltpu.VMEM((acc_rows, 16), jnp.float32),
            pltpu.SemaphoreType.DMA,
            pltpu.SemaphoreType.DMA,
            pltpu.SemaphoreType.DMA,
            pltpu.SemaphoreType.DMA,
        ],
        compiler_params=pltpu.CompilerParams(needs_layout_passes=False),
    )
    def sc_kernel(chat_hbm, fhat_hbm, nc_hbm, nf_hbm, codes_hbm, src_hbm,
                  dst_hbm, out_hbm, srcs_v, dsts_v, crows0, frows0, crows1,
                  frows1, nc_v, nf_v, codes_v, acc_v, sem_c0, sem_f0, sem_c1,
                  sem_f1):
        wid = lax.axis_index("s") * nc_cores + lax.axis_index("c")
        base = wid * e_per
        crows = (crows0, crows1)
        frows = (frows0, frows1)
        sem_c = (sem_c0, sem_c1)
        sem_f = (sem_f0, sem_f1)

        pltpu.sync_copy(src_hbm.at[pl.ds(base, e_per)], srcs_v)
        pltpu.sync_copy(dst_hbm.at[pl.ds(base, e_per)], dsts_v)
        pltpu.sync_copy(nc_hbm, nc_v)
        pltpu.sync_copy(nf_hbm, nf_v)
        pltpu.sync_copy(codes_hbm, codes_v)

        zeros16 = jnp.zeros((16,), jnp.float32)
        for b in range(acc_rows):
            acc_v[b, :] = zeros16

        lanes = lax.iota(jnp.int32, 16)
        ones16 = jnp.ones((16,), jnp.float32)

        def launch(cidx, b):
            off = cidx * ch
            cc = pltpu.async_copy(
                chat_hbm.at[srcs_v.at[pl.ds(off, ch)]], crows[b], sem_c[b]
            )
            cf = pltpu.async_copy(
                fhat_hbm.at[dsts_v.at[pl.ds(off, ch)]], frows[b], sem_f[b]
            )
            return cc, cf

        def wait(cidx, b):
            pltpu.make_async_copy(
                chat_hbm.at[srcs_v.at[pl.ds(cidx * ch, ch)]], crows[b],
                sem_c[b],
            ).wait()
            pltpu.make_async_copy(
                fhat_hbm.at[dsts_v.at[pl.ds(cidx * ch, ch)]], frows[b],
                sem_f[b],
            ).wait()

        def compute(cidx, b):
            cr = crows[b]
            fr = frows[b]

            def group_body(g, _):
                g16 = cidx * ch + g * 16
                s16 = srcs_v[pl.ds(g16, 16)]
                d16 = dsts_v[pl.ds(g16, 16)]
                acc = jnp.zeros((16,), jnp.float32)
                for j in range(16):
                    parts = [
                        cr[g * 16 + j, pl.ds(k * 16, 16)]
                        * fr[g * 16 + j, pl.ds(k * 16, 16)]
                        for k in range(d // 16)
                    ]
                    while len(parts) > 1:
                        parts = [
                            parts[i] + parts[i + 1]
                            for i in range(0, len(parts) - 1, 2)
                        ] + ([parts[-1]] if len(parts) % 2 else [])
                    red = _allreduce16(parts[0], lanes)
                    acc = jnp.where(lanes == j, red, acc)
                ncv = plsc.load_gather(nc_v, [s16])
                nfv = plsc.load_gather(nf_v, [d16])
                csv = plsc.load_gather(codes_v, [s16])
                cdv = plsc.load_gather(codes_v, [d16])
                scale = jnp.minimum(1.0, ncv * nfv * 1e8)
                ce = acc * scale
                acc_v[nbuckets, :] = acc_v[nbuckets, :] + ce
                for t in range(_NUM_TRIALS):
                    ps = (csv >> (2 * t)) & 3
                    pd = (cdv >> (2 * t)) & 3
                    m = ps == pd
                    bidx = ps + (4 * t)
                    plsc.addupdate_scatter(acc_v, [bidx, lanes], ce, mask=m)
                    plsc.addupdate_scatter(
                        acc_v, [bidx + (nbuckets + 1), lanes], ones16, mask=m
                    )
                return 0

            lax.fori_loop(0, n_groups, group_body, 0)

        launch(0, 0)
        launch(1, 1)

        def outer_body(k, _):
            c0 = 2 * k
            wait(c0, 0)
            compute(c0, 0)
            launch(c0 + 2, 0)
            wait(c0 + 1, 1)
            compute(c0 + 1, 1)

            @pl.when(c0 + 3 < n_chunks)
            def _():
                launch(c0 + 3, 1)

            return 0

        lax.fori_loop(0, (n_chunks - 1) // 2, outer_body, 0)
        wait(n_chunks - 1, 0)
        compute(n_chunks - 1, 0)
        pltpu.sync_copy(acc_v, out_hbm.at[wid])

    return sc_kernel


def kernel(x, edge_index, Wc, bc, We, be):
    n, d = x.shape
    e = edge_index.shape[1]
    info = plsc.get_sparse_core_info()
    nc_cores, ns = info.num_cores, info.num_subcores
    nw = nc_cores * ns
    assert e % nw == 0
    e_per = e // nw
    ch = 80
    assert e_per % ch == 0 and ch % 16 == 0

    dense = pl.pallas_call(
        _dense_body,
        out_shape=[
            jax.ShapeDtypeStruct((n, d), jnp.float32),
            jax.ShapeDtypeStruct((n, d), jnp.float32),
            jax.ShapeDtypeStruct((n, 1), jnp.float32),
            jax.ShapeDtypeStruct((n, 1), jnp.float32),
        ],
    )
    chat, fhat, ncn, nfn = dense(
        x,
        Wc.T,
        bc.reshape(1, d),
        We.T,
        be.reshape(1, d),
    )
    ncn = ncn.reshape(n)
    nfn = nfn.reshape(n)
    codes = jnp.asarray(_partition_codes(n))
    src = edge_index[0].astype(jnp.int32)
    dst = edge_index[1].astype(jnp.int32)

    sc = _make_sc_kernel(n, d, e_per, ch, nw, nc_cores)
    parts = sc(chat, fhat, ncn, nfn, codes, src, dst)

    tot = jnp.sum(parts, axis=(0, 2))
    nb = _NUM_TRIALS * _NUM_PARTITIONS
    sums = tot[0:nb].reshape(_NUM_TRIALS, _NUM_PARTITIONS)
    gsum = tot[nb]
    cnts = tot[nb + 1:2 * nb + 1].reshape(_NUM_TRIALS, _NUM_PARTITIONS)

    ei_whole = gsum / e
    pei = sums / jnp.maximum(cnts, 1.0)
    pvalid = cnts > 0
    nval = jnp.sum(pvalid, axis=1)
    avg = jnp.sum(jnp.where(pvalid, pei, 0.0), axis=1) / jnp.maximum(
        nval, 1
    ).astype(jnp.float32)
    tvalid = nval > 0
    any_valid = jnp.any(tvalid)
    min_pei_valid = jnp.min(jnp.where(tvalid, avg, jnp.inf))
    min_pei = jnp.where(any_valid, min_pei_valid, jnp.array(jnp.inf, jnp.float32))
    phi = jnp.where(
        any_valid,
        jnp.maximum(ei_whole - min_pei_valid, 0.0),
        jnp.maximum(ei_whole, 0.0),
    )
    return phi, ei_whole, min_pei


# speedup vs baseline: 604.6005x; 1.5409x over previous
import functools

import jax
import jax.numpy as jnp
import numpy as np
from jax import lax
from jax.experimental import pallas as pl
from jax.experimental.pallas import tpu as pltpu
from jax.experimental.pallas import tpu_sc as plsc

_NUM_PARTITIONS = 4
_NUM_TRIALS = 5

_GATHER_DN = lax.GatherDimensionNumbers(
    offset_dims=(), collapsed_slice_dims=(0,), start_index_map=(0,)
)


def _allreduce16(v, lanes):
    for sh in (1, 2, 4, 8):
        idx = (lanes ^ sh).reshape(16, 1)
        v = v + lax.gather(
            v, idx, _GATHER_DN, (1,),
            mode=lax.GatherScatterMode.PROMISE_IN_BOUNDS,
        )
    return v

_U32 = np.uint32


def _tf2x32(k1, k2, x1, x2):
    rot0 = (13, 15, 26, 6)
    rot1 = (17, 29, 16, 24)
    ks0 = _U32(k1)
    ks1 = _U32(k2)
    ks2 = _U32(ks0 ^ ks1 ^ _U32(0x1BD11BDA))
    a = (x1 + ks0).astype(_U32)
    b = (x2 + ks1).astype(_U32)

    def rnd(a, b, r):
        a = (a + b).astype(_U32)
        b = ((b << _U32(r)) | (b >> _U32(32 - r))).astype(_U32)
        return a, (a ^ b).astype(_U32)

    for r in rot0:
        a, b = rnd(a, b, r)
    a = (a + ks1).astype(_U32); b = (b + ks2 + _U32(1)).astype(_U32)
    for r in rot1:
        a, b = rnd(a, b, r)
    a = (a + ks2).astype(_U32); b = (b + ks0 + _U32(2)).astype(_U32)
    for r in rot0:
        a, b = rnd(a, b, r)
    a = (a + ks0).astype(_U32); b = (b + ks1 + _U32(3)).astype(_U32)
    for r in rot1:
        a, b = rnd(a, b, r)
    a = (a + ks1).astype(_U32); b = (b + ks2 + _U32(4)).astype(_U32)
    for r in rot0:
        a, b = rnd(a, b, r)
    a = (a + ks2).astype(_U32); b = (b + ks0 + _U32(5)).astype(_U32)
    return a, b


def _np_permutation(key, n):
    x = np.arange(n)
    num_rounds = int(
        np.ceil(3 * np.log(max(1, n)) / np.log(2**32 - 1))
    )
    for _ in range(num_rounds):
        a, b = _tf2x32(
            key[0], key[1], np.array([0, 0], _U32), np.array([0, 1], _U32)
        )
        key, subkey = (a[0], b[0]), (a[1], b[1])
        ra, rb = _tf2x32(
            subkey[0], subkey[1],
            np.zeros(n, _U32), np.arange(n, dtype=_U32),
        )
        order = np.argsort((ra ^ rb).astype(_U32), kind="stable")
        x = x[order]
    return x


_CODES_CACHE = {}


def _partition_codes(n: int) -> np.ndarray:
    if n not in _CODES_CACHE:
        old_err = np.seterr(over="ignore")
        try:
            base = (_U32(0), _U32(42))
            psize = n // _NUM_PARTITIONS
            part_of_pos = np.minimum(
                np.arange(n) // psize, _NUM_PARTITIONS - 1
            ).astype(np.int32)
            code = np.zeros((n,), np.int32)
            for t in range(_NUM_TRIALS):
                ka, kb = _tf2x32(
                    base[0], base[1], np.array([0], _U32),
                    np.array([t], _U32),
                )
                perm = _np_permutation((ka[0], kb[0]), n)
                part = np.zeros((n,), np.int32)
                part[perm] = part_of_pos
                code |= part << (2 * t)
        finally:
            np.seterr(**old_err)
        _CODES_CACHE[n] = code
    return _CODES_CACHE[n]


def _dense_body(x_ref, wct_ref, bc_ref, wet_ref, be_ref,
                chat_ref, fhat_ref, nc_ref, nf_ref):
    x = x_ref[...]
    dn = (((1,), (0,)), ((), ()))
    c = jnp.tanh(
        lax.dot_general(x, wct_ref[...], dn, preferred_element_type=jnp.float32)
        + bc_ref[...]
    )
    f = jnp.tanh(
        lax.dot_general(x, wet_ref[...], dn, preferred_element_type=jnp.float32)
        + be_ref[...]
    )
    nc = jnp.sqrt(jnp.sum(c * c, axis=1, keepdims=True))
    nf = jnp.sqrt(jnp.sum(f * f, axis=1, keepdims=True))
    chat = c * jnp.where(nc > 0, 1.0 / nc, 0.0)
    fhat = f * jnp.where(nf > 0, 1.0 / nf, 0.0)
    chat_ref[...] = chat.astype(jnp.bfloat16)
    fhat_ref[...] = fhat.astype(jnp.bfloat16)
    nc_ref[...] = nc
    nf_ref[...] = nf


def _make_sc_kernel(n: int, d: int, e_per: int, ch: int, nw: int, nc_cores: int):
    mesh = plsc.VectorSubcoreMesh(core_axis_name="c", subcore_axis_name="s")
    n_chunks = e_per // ch
    n_groups = ch // 16
    nbuckets = _NUM_TRIALS * _NUM_PARTITIONS
    acc_rows = 2 * nbuckets + 1

    assert n_chunks % 2 == 1, "pipeline epilogue below assumes odd n_chunks"

    @functools.partial(
        pl.kernel,
        mesh=mesh,
        out_type=jax.ShapeDtypeStruct((nw, acc_rows, 16), jnp.float32),
        scratch_types=[
            pltpu.VMEM((e_per,), jnp.int32),
            pltpu.VMEM((e_per,), jnp.int32),
            pltpu.VMEM((ch, d), jnp.bfloat16),
            pltpu.VMEM((ch, d), jnp.bfloat16),
            pltpu.VMEM((ch, d), jnp.bfloat16),
            pltpu.VMEM((ch, d), jnp.bfloat16),
            pltpu.VMEM((n,), jnp.float32),
            pltpu.VMEM((n,), jnp.float32),
            pltpu.VMEM((n,), jnp.int32),
            pltpu.VMEM((acc_rows, 16), jnp.float32),
            pltpu.SemaphoreType.DMA,
            pltpu.SemaphoreType.DMA,
            pltpu.SemaphoreType.DMA,
            pltpu.SemaphoreType.DMA,
        ],
        compiler_params=pltpu.CompilerParams(
            needs_layout_passes=False, use_tc_tiling_on_sc=False
        ),
    )
    def sc_kernel(chat_hbm, fhat_hbm, nc_hbm, nf_hbm, codes_hbm, src_hbm,
                  dst_hbm, out_hbm, srcs_v, dsts_v, crows0, frows0, crows1,
                  frows1, nc_v, nf_v, codes_v, acc_v, sem_c0, sem_f0, sem_c1,
                  sem_f1):
        wid = lax.axis_index("s") * nc_cores + lax.axis_index("c")
        base = wid * e_per
        crows = (crows0, crows1)
        frows = (frows0, frows1)
        sem_c = (sem_c0, sem_c1)
        sem_f = (sem_f0, sem_f1)

        pltpu.sync_copy(src_hbm.at[pl.ds(base, e_per)], srcs_v)
        pltpu.sync_copy(dst_hbm.at[pl.ds(base, e_per)], dsts_v)
        pltpu.sync_copy(nc_hbm, nc_v)
        pltpu.sync_copy(nf_hbm, nf_v)
        pltpu.sync_copy(codes_hbm, codes_v)

        zeros16 = jnp.zeros((16,), jnp.float32)
        for b in range(acc_rows):
            acc_v[b, :] = zeros16

        lanes = lax.iota(jnp.int32, 16)
        ones16 = jnp.ones((16,), jnp.float32)

        def launch(cidx, b):
            off = cidx * ch
            cc = pltpu.async_copy(
                chat_hbm.at[srcs_v.at[pl.ds(off, ch)]], crows[b], sem_c[b]
            )
            cf = pltpu.async_copy(
                fhat_hbm.at[dsts_v.at[pl.ds(off, ch)]], frows[b], sem_f[b]
            )
            return cc, cf

        def wait(cidx, b):
            pltpu.make_async_copy(
                chat_hbm.at[srcs_v.at[pl.ds(cidx * ch, ch)]], crows[b],
                sem_c[b],
            ).wait()
            pltpu.make_async_copy(
                fhat_hbm.at[dsts_v.at[pl.ds(cidx * ch, ch)]], frows[b],
                sem_f[b],
            ).wait()

        def compute(cidx, b):
            cr = crows[b]
            fr = frows[b]

            def group_body(g, _):
                g16 = cidx * ch + g * 16
                s16 = srcs_v[pl.ds(g16, 16)]
                d16 = dsts_v[pl.ds(g16, 16)]
                acc = jnp.zeros((16,), jnp.float32)
                for j in range(16):
                    parts = []
                    for k in range(d // 32):
                        cv = cr[g * 16 + j, pl.ds(k * 32, 32)]
                        fv = fr[g * 16 + j, pl.ds(k * 32, 32)]
                        ca, cb = plsc.unpack(
                            cv, format=plsc.PackFormat.INTERLEAVED
                        )
                        fa, fb = plsc.unpack(
                            fv, format=plsc.PackFormat.INTERLEAVED
                        )
                        parts.append(ca * fa)
                        parts.append(cb * fb)
                    while len(parts) > 1:
                        parts = [
                            parts[i] + parts[i + 1]
                            for i in range(0, len(parts) - 1, 2)
                        ] + ([parts[-1]] if len(parts) % 2 else [])
                    red = _allreduce16(parts[0], lanes)
                    acc = jnp.where(lanes == j, red, acc)
                ncv = plsc.load_gather(nc_v, [s16])
                nfv = plsc.load_gather(nf_v, [d16])
                csv = plsc.load_gather(codes_v, [s16])
                cdv = plsc.load_gather(codes_v, [d16])
                scale = jnp.minimum(1.0, ncv * nfv * 1e8)
                ce = acc * scale
                acc_v[nbuckets, :] = acc_v[nbuckets, :] + ce
                for t in range(_NUM_TRIALS):
                    ps = (csv >> (2 * t)) & 3
                    pd = (cdv >> (2 * t)) & 3
                    m = ps == pd
                    bidx = ps + (4 * t)
                    plsc.addupdate_scatter(acc_v, [bidx, lanes], ce, mask=m)
                    plsc.addupdate_scatter(
                        acc_v, [bidx + (nbuckets + 1), lanes], ones16, mask=m
                    )
                return 0

            lax.fori_loop(0, n_groups, group_body, 0)

        launch(0, 0)
        launch(1, 1)

        def outer_body(k, _):
            c0 = 2 * k
            wait(c0, 0)
            compute(c0, 0)
            launch(c0 + 2, 0)
            wait(c0 + 1, 1)
            compute(c0 + 1, 1)

            @pl.when(c0 + 3 < n_chunks)
            def _():
                launch(c0 + 3, 1)

            return 0

        lax.fori_loop(0, (n_chunks - 1) // 2, outer_body, 0)
        wait(n_chunks - 1, 0)
        compute(n_chunks - 1, 0)
        pltpu.sync_copy(acc_v, out_hbm.at[wid])

    return sc_kernel


def kernel(x, edge_index, Wc, bc, We, be):
    n, d = x.shape
    e = edge_index.shape[1]
    info = plsc.get_sparse_core_info()
    nc_cores, ns = info.num_cores, info.num_subcores
    nw = nc_cores * ns
    assert e % nw == 0
    e_per = e // nw
    ch = 80
    assert e_per % ch == 0 and ch % 16 == 0

    dense = pl.pallas_call(
        _dense_body,
        out_shape=[
            jax.ShapeDtypeStruct((n, d), jnp.bfloat16),
            jax.ShapeDtypeStruct((n, d), jnp.bfloat16),
            jax.ShapeDtypeStruct((n, 1), jnp.float32),
            jax.ShapeDtypeStruct((n, 1), jnp.float32),
        ],
    )
    chat, fhat, ncn, nfn = dense(
        x,
        Wc.T,
        bc.reshape(1, d),
        We.T,
        be.reshape(1, d),
    )
    ncn = ncn.reshape(n)
    nfn = nfn.reshape(n)
    codes = jnp.asarray(_partition_codes(n))
    src = edge_index[0].astype(jnp.int32)
    dst = edge_index[1].astype(jnp.int32)

    sc = _make_sc_kernel(n, d, e_per, ch, nw, nc_cores)
    parts = sc(chat, fhat, ncn, nfn, codes, src, dst)

    tot = jnp.sum(parts, axis=(0, 2))
    nb = _NUM_TRIALS * _NUM_PARTITIONS
    sums = tot[0:nb].reshape(_NUM_TRIALS, _NUM_PARTITIONS)
    gsum = tot[nb]
    cnts = tot[nb + 1:2 * nb + 1].reshape(_NUM_TRIALS, _NUM_PARTITIONS)

    ei_whole = gsum / e
    pei = sums / jnp.maximum(cnts, 1.0)
    pvalid = cnts > 0
    nval = jnp.sum(pvalid, axis=1)
    avg = jnp.sum(jnp.where(pvalid, pei, 0.0), axis=1) / jnp.maximum(
        nval, 1
    ).astype(jnp.float32)
    tvalid = nval > 0
    any_valid = jnp.any(tvalid)
    min_pei_valid = jnp.min(jnp.where(tvalid, avg, jnp.inf))
    min_pei = jnp.where(any_valid, min_pei_valid, jnp.array(jnp.inf, jnp.float32))
    phi = jnp.where(
        any_valid,
        jnp.maximum(ei_whole - min_pei_valid, 0.0),
        jnp.maximum(ei_whole, 0.0),
    )
    return phi, ei_whole, min_pei


# speedup vs baseline: 624.9090x; 1.0336x over previous
import functools

import jax
import jax.numpy as jnp
import numpy as np
from jax import lax
from jax.experimental import pallas as pl
from jax.experimental.pallas import tpu as pltpu
from jax.experimental.pallas import tpu_sc as plsc

_NUM_PARTITIONS = 4
_NUM_TRIALS = 5

_GATHER_DN = lax.GatherDimensionNumbers(
    offset_dims=(), collapsed_slice_dims=(0,), start_index_map=(0,)
)


def _allreduce16(v, lanes):
    for sh in (1, 2, 4, 8):
        idx = (lanes ^ sh).reshape(16, 1)
        v = v + lax.gather(
            v, idx, _GATHER_DN, (1,),
            mode=lax.GatherScatterMode.PROMISE_IN_BOUNDS,
        )
    return v

_U32 = np.uint32


def _tf2x32(k1, k2, x1, x2):
    rot0 = (13, 15, 26, 6)
    rot1 = (17, 29, 16, 24)
    ks0 = _U32(k1)
    ks1 = _U32(k2)
    ks2 = _U32(ks0 ^ ks1 ^ _U32(0x1BD11BDA))
    a = (x1 + ks0).astype(_U32)
    b = (x2 + ks1).astype(_U32)

    def rnd(a, b, r):
        a = (a + b).astype(_U32)
        b = ((b << _U32(r)) | (b >> _U32(32 - r))).astype(_U32)
        return a, (a ^ b).astype(_U32)

    for r in rot0:
        a, b = rnd(a, b, r)
    a = (a + ks1).astype(_U32); b = (b + ks2 + _U32(1)).astype(_U32)
    for r in rot1:
        a, b = rnd(a, b, r)
    a = (a + ks2).astype(_U32); b = (b + ks0 + _U32(2)).astype(_U32)
    for r in rot0:
        a, b = rnd(a, b, r)
    a = (a + ks0).astype(_U32); b = (b + ks1 + _U32(3)).astype(_U32)
    for r in rot1:
        a, b = rnd(a, b, r)
    a = (a + ks1).astype(_U32); b = (b + ks2 + _U32(4)).astype(_U32)
    for r in rot0:
        a, b = rnd(a, b, r)
    a = (a + ks2).astype(_U32); b = (b + ks0 + _U32(5)).astype(_U32)
    return a, b


def _np_permutation(key, n):
    x = np.arange(n)
    num_rounds = int(
        np.ceil(3 * np.log(max(1, n)) / np.log(2**32 - 1))
    )
    for _ in range(num_rounds):
        a, b = _tf2x32(
            key[0], key[1], np.array([0, 0], _U32), np.array([0, 1], _U32)
        )
        key, subkey = (a[0], b[0]), (a[1], b[1])
        ra, rb = _tf2x32(
            subkey[0], subkey[1],
            np.zeros(n, _U32), np.arange(n, dtype=_U32),
        )
        order = np.argsort((ra ^ rb).astype(_U32), kind="stable")
        x = x[order]
    return x


_CODES_CACHE = {}


def _partition_codes(n: int) -> np.ndarray:
    if n not in _CODES_CACHE:
        old_err = np.seterr(over="ignore")
        try:
            base = (_U32(0), _U32(42))
            psize = n // _NUM_PARTITIONS
            part_of_pos = np.minimum(
                np.arange(n) // psize, _NUM_PARTITIONS - 1
            ).astype(np.int32)
            code = np.zeros((n,), np.int32)
            for t in range(_NUM_TRIALS):
                ka, kb = _tf2x32(
                    base[0], base[1], np.array([0], _U32),
                    np.array([t], _U32),
                )
                perm = _np_permutation((ka[0], kb[0]), n)
                part = np.zeros((n,), np.int32)
                part[perm] = part_of_pos
                code |= part << (2 * t)
        finally:
            np.seterr(**old_err)
        _CODES_CACHE[n] = code
    return _CODES_CACHE[n]


def _dense_body(x_ref, wct_ref, bc_ref, wet_ref, be_ref,
                chat_ref, fhat_ref, nc_ref, nf_ref):
    x = x_ref[...]
    dn = (((1,), (1,)), ((), ()))
    c = jnp.tanh(
        lax.dot_general(x, wct_ref[...], dn, preferred_element_type=jnp.float32)
        + bc_ref[...]
    )
    f = jnp.tanh(
        lax.dot_general(x, wet_ref[...], dn, preferred_element_type=jnp.float32)
        + be_ref[...]
    )
    nc = jnp.sqrt(jnp.sum(c * c, axis=1, keepdims=True))
    nf = jnp.sqrt(jnp.sum(f * f, axis=1, keepdims=True))
    chat = c * jnp.where(nc > 0, 1.0 / nc, 0.0)
    fhat = f * jnp.where(nf > 0, 1.0 / nf, 0.0)
    chat_ref[...] = chat.astype(jnp.bfloat16)
    fhat_ref[...] = fhat.astype(jnp.bfloat16)
    nc_ref[...] = nc
    nf_ref[...] = nf


def _final_body(parts_ref, out_ref, *, e):
    nb = _NUM_TRIALS * _NUM_PARTITIONS
    tot = jnp.sum(parts_ref[...], axis=(0, 2))
    sums = tot[0:nb].reshape(_NUM_TRIALS, _NUM_PARTITIONS)
    gsum = tot[nb]
    cnts = tot[nb + 1:2 * nb + 1].reshape(_NUM_TRIALS, _NUM_PARTITIONS)

    ei_whole = gsum / e
    pei = sums / jnp.maximum(cnts, 1.0)
    pvalid = cnts > 0
    nval = jnp.sum(pvalid, axis=1)
    avg = jnp.sum(jnp.where(pvalid, pei, 0.0), axis=1) / jnp.maximum(
        nval, 1
    ).astype(jnp.float32)
    tvalid = nval > 0
    any_valid = jnp.any(tvalid)
    min_pei_valid = jnp.min(jnp.where(tvalid, avg, jnp.inf))
    min_pei = jnp.where(
        any_valid, min_pei_valid, jnp.array(jnp.inf, jnp.float32)
    )
    phi = jnp.where(
        any_valid,
        jnp.maximum(ei_whole - min_pei_valid, 0.0),
        jnp.maximum(ei_whole, 0.0),
    )
    out_ref[...] = jnp.concatenate(
        [phi.reshape(1, 1), ei_whole.reshape(1, 1), min_pei.reshape(1, 1)],
        axis=1,
    )


def _make_sc_kernel(n: int, d: int, e_per: int, ch: int, nw: int, nc_cores: int):
    mesh = plsc.VectorSubcoreMesh(core_axis_name="c", subcore_axis_name="s")
    n_chunks = e_per // ch
    n_groups = ch // 16
    nbuckets = _NUM_TRIALS * _NUM_PARTITIONS
    acc_rows = 2 * nbuckets + 1

    assert n_chunks % 2 == 1, "pipeline epilogue below assumes odd n_chunks"

    @functools.partial(
        pl.kernel,
        mesh=mesh,
        out_type=jax.ShapeDtypeStruct((nw, acc_rows, 16), jnp.float32),
        scratch_types=[
            pltpu.VMEM((e_per,), jnp.int32),
            pltpu.VMEM((e_per,), jnp.int32),
            pltpu.VMEM((ch, d), jnp.bfloat16),
            pltpu.VMEM((ch, d), jnp.bfloat16),
            pltpu.VMEM((ch, d), jnp.bfloat16),
            pltpu.VMEM((ch, d), jnp.bfloat16),
            pltpu.VMEM((n,), jnp.float32),
            pltpu.VMEM((n,), jnp.float32),
            pltpu.VMEM((n,), jnp.int32),
            pltpu.VMEM((acc_rows, 16), jnp.float32),
            pltpu.SemaphoreType.DMA,
            pltpu.SemaphoreType.DMA,
            pltpu.SemaphoreType.DMA,
            pltpu.SemaphoreType.DMA,
        ],
        compiler_params=pltpu.CompilerParams(
            needs_layout_passes=False, use_tc_tiling_on_sc=False
        ),
    )
    def sc_kernel(chat_hbm, fhat_hbm, nc_hbm, nf_hbm, codes_hbm, src_hbm,
                  dst_hbm, out_hbm, srcs_v, dsts_v, crows0, frows0, crows1,
                  frows1, nc_v, nf_v, codes_v, acc_v, sem_c0, sem_f0, sem_c1,
                  sem_f1):
        wid = lax.axis_index("s") * nc_cores + lax.axis_index("c")
        base = wid * e_per
        crows = (crows0, crows1)
        frows = (frows0, frows1)
        sem_c = (sem_c0, sem_c1)
        sem_f = (sem_f0, sem_f1)

        pltpu.sync_copy(src_hbm.at[pl.ds(base, e_per)], srcs_v)
        pltpu.sync_copy(dst_hbm.at[pl.ds(base, e_per)], dsts_v)
        pltpu.sync_copy(nc_hbm, nc_v)
        pltpu.sync_copy(nf_hbm, nf_v)
        pltpu.sync_copy(codes_hbm, codes_v)

        zeros16 = jnp.zeros((16,), jnp.float32)
        for b in range(acc_rows):
            acc_v[b, :] = zeros16

        lanes = lax.iota(jnp.int32, 16)
        ones16 = jnp.ones((16,), jnp.float32)

        def launch(cidx, b):
            off = cidx * ch
            cc = pltpu.async_copy(
                chat_hbm.at[srcs_v.at[pl.ds(off, ch)]], crows[b], sem_c[b]
            )
            cf = pltpu.async_copy(
                fhat_hbm.at[dsts_v.at[pl.ds(off, ch)]], frows[b], sem_f[b]
            )
            return cc, cf

        def wait(cidx, b):
            pltpu.make_async_copy(
                chat_hbm.at[srcs_v.at[pl.ds(cidx * ch, ch)]], crows[b],
                sem_c[b],
            ).wait()
            pltpu.make_async_copy(
                fhat_hbm.at[dsts_v.at[pl.ds(cidx * ch, ch)]], frows[b],
                sem_f[b],
            ).wait()

        def compute(cidx, b):
            cr = crows[b]
            fr = frows[b]

            def group_body(g, _):
                g16 = cidx * ch + g * 16
                s16 = srcs_v[pl.ds(g16, 16)]
                d16 = dsts_v[pl.ds(g16, 16)]
                acc = jnp.zeros((16,), jnp.float32)
                for j in range(16):
                    parts = []
                    for k in range(d // 32):
                        cv = cr[g * 16 + j, pl.ds(k * 32, 32)]
                        fv = fr[g * 16 + j, pl.ds(k * 32, 32)]
                        ca, cb = plsc.unpack(
                            cv, format=plsc.PackFormat.INTERLEAVED
                        )
                        fa, fb = plsc.unpack(
                            fv, format=plsc.PackFormat.INTERLEAVED
                        )
                        parts.append(ca * fa)
                        parts.append(cb * fb)
                    while len(parts) > 1:
                        parts = [
                            parts[i] + parts[i + 1]
                            for i in range(0, len(parts) - 1, 2)
                        ] + ([parts[-1]] if len(parts) % 2 else [])
                    red = _allreduce16(parts[0], lanes)
                    acc = jnp.where(lanes == j, red, acc)
                ncv = plsc.load_gather(nc_v, [s16])
                nfv = plsc.load_gather(nf_v, [d16])
                csv = plsc.load_gather(codes_v, [s16])
                cdv = plsc.load_gather(codes_v, [d16])
                scale = jnp.minimum(1.0, ncv * nfv * 1e8)
                ce = acc * scale
                acc_v[nbuckets, :] = acc_v[nbuckets, :] + ce
                for t in range(_NUM_TRIALS):
                    ps = (csv >> (2 * t)) & 3
                    pd = (cdv >> (2 * t)) & 3
                    m = ps == pd
                    bidx = ps + (4 * t)
                    plsc.addupdate_scatter(acc_v, [bidx, lanes], ce, mask=m)
                    plsc.addupdate_scatter(
                        acc_v, [bidx + (nbuckets + 1), lanes], ones16, mask=m
                    )
                return 0

            lax.fori_loop(0, n_groups, group_body, 0)

        launch(0, 0)
        launch(1, 1)

        def outer_body(k, _):
            c0 = 2 * k
            wait(c0, 0)
            compute(c0, 0)
            launch(c0 + 2, 0)
            wait(c0 + 1, 1)
            compute(c0 + 1, 1)

            @pl.when(c0 + 3 < n_chunks)
            def _():
                launch(c0 + 3, 1)

            return 0

        lax.fori_loop(0, (n_chunks - 1) // 2, outer_body, 0)
        wait(n_chunks - 1, 0)
        compute(n_chunks - 1, 0)
        pltpu.sync_copy(acc_v, out_hbm.at[wid])

    return sc_kernel


def kernel(x, edge_index, Wc, bc, We, be):
    n, d = x.shape
    e = edge_index.shape[1]
    info = plsc.get_sparse_core_info()
    nc_cores, ns = info.num_cores, info.num_subcores
    nw = nc_cores * ns
    assert e % nw == 0
    e_per = e // nw
    ch = 80
    assert e_per % ch == 0 and ch % 16 == 0

    dense = pl.pallas_call(
        _dense_body,
        out_shape=[
            jax.ShapeDtypeStruct((n, d), jnp.bfloat16),
            jax.ShapeDtypeStruct((n, d), jnp.bfloat16),
            jax.ShapeDtypeStruct((n, 1), jnp.float32),
            jax.ShapeDtypeStruct((n, 1), jnp.float32),
        ],
    )
    chat, fhat, ncn, nfn = dense(
        x,
        Wc,
        bc.reshape(1, d),
        We,
        be.reshape(1, d),
    )
    ncn = ncn.reshape(n)
    nfn = nfn.reshape(n)
    codes = jnp.asarray(_partition_codes(n))
    src = edge_index[0].astype(jnp.int32)
    dst = edge_index[1].astype(jnp.int32)

    sc = _make_sc_kernel(n, d, e_per, ch, nw, nc_cores)
    parts = sc(chat, fhat, ncn, nfn, codes, src, dst)

    final = pl.pallas_call(
        functools.partial(_final_body, e=e),
        out_shape=jax.ShapeDtypeStruct((1, 3), jnp.float32),
    )
    out = final(parts)
    phi = out[0, 0]
    ei_whole = out[0, 1]
    min_pei = out[0, 2]
    return phi, ei_whole, min_pei


# speedup vs baseline: 652.3081x; 1.0438x over previous
import functools

import jax
import jax.numpy as jnp
import numpy as np
from jax import lax
from jax.experimental import pallas as pl
from jax.experimental.pallas import tpu as pltpu
from jax.experimental.pallas import tpu_sc as plsc

_NUM_PARTITIONS = 4
_NUM_TRIALS = 5

_GATHER_DN = lax.GatherDimensionNumbers(
    offset_dims=(), collapsed_slice_dims=(0,), start_index_map=(0,)
)


def _allreduce16(v, lanes):
    for sh in (1, 2, 4, 8):
        idx = (lanes ^ sh).reshape(16, 1)
        v = v + lax.gather(
            v, idx, _GATHER_DN, (1,),
            mode=lax.GatherScatterMode.PROMISE_IN_BOUNDS,
        )
    return v

_U32 = np.uint32


def _tf2x32(k1, k2, x1, x2):
    rot0 = (13, 15, 26, 6)
    rot1 = (17, 29, 16, 24)
    ks0 = _U32(k1)
    ks1 = _U32(k2)
    ks2 = _U32(ks0 ^ ks1 ^ _U32(0x1BD11BDA))
    a = (x1 + ks0).astype(_U32)
    b = (x2 + ks1).astype(_U32)

    def rnd(a, b, r):
        a = (a + b).astype(_U32)
        b = ((b << _U32(r)) | (b >> _U32(32 - r))).astype(_U32)
        return a, (a ^ b).astype(_U32)

    for r in rot0:
        a, b = rnd(a, b, r)
    a = (a + ks1).astype(_U32); b = (b + ks2 + _U32(1)).astype(_U32)
    for r in rot1:
        a, b = rnd(a, b, r)
    a = (a + ks2).astype(_U32); b = (b + ks0 + _U32(2)).astype(_U32)
    for r in rot0:
        a, b = rnd(a, b, r)
    a = (a + ks0).astype(_U32); b = (b + ks1 + _U32(3)).astype(_U32)
    for r in rot1:
        a, b = rnd(a, b, r)
    a = (a + ks1).astype(_U32); b = (b + ks2 + _U32(4)).astype(_U32)
    for r in rot0:
        a, b = rnd(a, b, r)
    a = (a + ks2).astype(_U32); b = (b + ks0 + _U32(5)).astype(_U32)
    return a, b


def _np_permutation(key, n):
    x = np.arange(n)
    num_rounds = int(
        np.ceil(3 * np.log(max(1, n)) / np.log(2**32 - 1))
    )
    for _ in range(num_rounds):
        a, b = _tf2x32(
            key[0], key[1], np.array([0, 0], _U32), np.array([0, 1], _U32)
        )
        key, subkey = (a[0], b[0]), (a[1], b[1])
        ra, rb = _tf2x32(
            subkey[0], subkey[1],
            np.zeros(n, _U32), np.arange(n, dtype=_U32),
        )
        order = np.argsort((ra ^ rb).astype(_U32), kind="stable")
        x = x[order]
    return x


_CODES_CACHE = {}


def _partition_codes(n: int) -> np.ndarray:
    if n not in _CODES_CACHE:
        old_err = np.seterr(over="ignore")
        try:
            base = (_U32(0), _U32(42))
            psize = n // _NUM_PARTITIONS
            part_of_pos = np.minimum(
                np.arange(n) // psize, _NUM_PARTITIONS - 1
            ).astype(np.int32)
            code = np.zeros((n,), np.int32)
            for t in range(_NUM_TRIALS):
                ka, kb = _tf2x32(
                    base[0], base[1], np.array([0], _U32),
                    np.array([t], _U32),
                )
                perm = _np_permutation((ka[0], kb[0]), n)
                part = np.zeros((n,), np.int32)
                part[perm] = part_of_pos
                code |= part << (2 * t)
        finally:
            np.seterr(**old_err)
        _CODES_CACHE[n] = code
    return _CODES_CACHE[n]


def _dense_body(x_ref, wct_ref, bc_ref, wet_ref, be_ref,
                chat_ref, fhat_ref, nc_ref, nf_ref):
    x = x_ref[...]
    dn = (((1,), (1,)), ((), ()))
    c = jnp.tanh(
        lax.dot_general(x, wct_ref[...], dn, preferred_element_type=jnp.float32)
        + bc_ref[...]
    )
    f = jnp.tanh(
        lax.dot_general(x, wet_ref[...], dn, preferred_element_type=jnp.float32)
        + be_ref[...]
    )
    nc = jnp.sqrt(jnp.sum(c * c, axis=1, keepdims=True))
    nf = jnp.sqrt(jnp.sum(f * f, axis=1, keepdims=True))
    chat = c * jnp.where(nc > 0, 1.0 / nc, 0.0)
    fhat = f * jnp.where(nf > 0, 1.0 / nf, 0.0)
    chat_ref[...] = chat.astype(jnp.bfloat16)
    fhat_ref[...] = fhat.astype(jnp.bfloat16)
    nc_ref[...] = nc
    nf_ref[...] = nf


def _final_body(parts_ref, out_ref, *, e):
    nb = _NUM_TRIALS * _NUM_PARTITIONS
    tot = jnp.sum(parts_ref[...], axis=(0, 2))
    sums = tot[0:nb].reshape(_NUM_TRIALS, _NUM_PARTITIONS)
    gsum = tot[nb]
    cnts = tot[nb + 1:2 * nb + 1].reshape(_NUM_TRIALS, _NUM_PARTITIONS)

    ei_whole = gsum / e
    pei = sums / jnp.maximum(cnts, 1.0)
    pvalid = cnts > 0
    nval = jnp.sum(pvalid, axis=1)
    avg = jnp.sum(jnp.where(pvalid, pei, 0.0), axis=1) / jnp.maximum(
        nval, 1
    ).astype(jnp.float32)
    tvalid = nval > 0
    any_valid = jnp.any(tvalid)
    min_pei_valid = jnp.min(jnp.where(tvalid, avg, jnp.inf))
    min_pei = jnp.where(
        any_valid, min_pei_valid, jnp.array(jnp.inf, jnp.float32)
    )
    phi = jnp.where(
        any_valid,
        jnp.maximum(ei_whole - min_pei_valid, 0.0),
        jnp.maximum(ei_whole, 0.0),
    )
    out_ref[...] = jnp.concatenate(
        [phi.reshape(1, 1), ei_whole.reshape(1, 1), min_pei.reshape(1, 1)],
        axis=1,
    )


def _make_sc_kernel(n: int, d: int, e_per: int, ch: int, nw: int, nc_cores: int):
    mesh = plsc.VectorSubcoreMesh(core_axis_name="c", subcore_axis_name="s")
    n_chunks = e_per // ch
    n_groups = ch // 16
    nbuckets = _NUM_TRIALS * _NUM_PARTITIONS
    acc_rows = 2 * nbuckets + 1

    assert n_chunks % 2 == 1, "pipeline epilogue below assumes odd n_chunks"

    @functools.partial(
        pl.kernel,
        mesh=mesh,
        out_type=jax.ShapeDtypeStruct((nw, acc_rows, 16), jnp.float32),
        scratch_types=[
            pltpu.VMEM((e_per,), jnp.int32),
            pltpu.VMEM((e_per,), jnp.int32),
            pltpu.VMEM((ch, d), jnp.bfloat16),
            pltpu.VMEM((ch, d), jnp.bfloat16),
            pltpu.VMEM((ch, d), jnp.bfloat16),
            pltpu.VMEM((ch, d), jnp.bfloat16),
            pltpu.VMEM((n,), jnp.float32),
            pltpu.VMEM((n,), jnp.float32),
            pltpu.VMEM((n,), jnp.int32),
            pltpu.VMEM((acc_rows, 16), jnp.float32),
            pltpu.SemaphoreType.DMA,
            pltpu.SemaphoreType.DMA,
            pltpu.SemaphoreType.DMA,
            pltpu.SemaphoreType.DMA,
        ],
        compiler_params=pltpu.CompilerParams(
            needs_layout_passes=False, use_tc_tiling_on_sc=False
        ),
    )
    def sc_kernel(chat_hbm, fhat_hbm, nc_hbm, nf_hbm, codes_hbm, src_hbm,
                  dst_hbm, out_hbm, srcs_v, dsts_v, crows0, frows0, crows1,
                  frows1, nc_v, nf_v, codes_v, acc_v, sem_c0, sem_f0, sem_c1,
                  sem_f1):
        wid = lax.axis_index("s") * nc_cores + lax.axis_index("c")
        base = wid * e_per
        crows = (crows0, crows1)
        frows = (frows0, frows1)
        sem_c = (sem_c0, sem_c1)
        sem_f = (sem_f0, sem_f1)

        pltpu.sync_copy(src_hbm.at[pl.ds(base, e_per)], srcs_v)
        pltpu.sync_copy(dst_hbm.at[pl.ds(base, e_per)], dsts_v)
        pltpu.sync_copy(nc_hbm, nc_v)
        pltpu.sync_copy(nf_hbm, nf_v)
        pltpu.sync_copy(codes_hbm, codes_v)

        zeros16 = jnp.zeros((16,), jnp.float32)
        for b in range(acc_rows):
            acc_v[b, :] = zeros16

        lanes = lax.iota(jnp.int32, 16)
        ones16 = jnp.ones((16,), jnp.float32)

        def launch(cidx, b):
            off = cidx * ch
            cc = pltpu.async_copy(
                chat_hbm.at[srcs_v.at[pl.ds(off, ch)]], crows[b], sem_c[b]
            )
            cf = pltpu.async_copy(
                fhat_hbm.at[dsts_v.at[pl.ds(off, ch)]], frows[b], sem_f[b]
            )
            return cc, cf

        def wait(cidx, b):
            pltpu.make_async_copy(
                chat_hbm.at[srcs_v.at[pl.ds(cidx * ch, ch)]], crows[b],
                sem_c[b],
            ).wait()
            pltpu.make_async_copy(
                fhat_hbm.at[dsts_v.at[pl.ds(cidx * ch, ch)]], frows[b],
                sem_f[b],
            ).wait()

        def compute(cidx, b):
            cr = crows[b]
            fr = frows[b]

            def group_body(g, _):
                g16 = cidx * ch + g * 16
                s16 = srcs_v[pl.ds(g16, 16)]
                d16 = dsts_v[pl.ds(g16, 16)]
                acc = jnp.zeros((16,), jnp.float32)
                for j in range(16):
                    parts = []
                    for k in range(d // 32):
                        cv = cr[g * 16 + j, pl.ds(k * 32, 32)]
                        fv = fr[g * 16 + j, pl.ds(k * 32, 32)]
                        pa, pb = plsc.unpack(
                            cv * fv, format=plsc.PackFormat.INTERLEAVED
                        )
                        parts.append(pa)
                        parts.append(pb)
                    while len(parts) > 1:
                        parts = [
                            parts[i] + parts[i + 1]
                            for i in range(0, len(parts) - 1, 2)
                        ] + ([parts[-1]] if len(parts) % 2 else [])
                    red = _allreduce16(parts[0], lanes)
                    acc = jnp.where(lanes == j, red, acc)
                ncv = plsc.load_gather(nc_v, [s16])
                nfv = plsc.load_gather(nf_v, [d16])
                csv = plsc.load_gather(codes_v, [s16])
                cdv = plsc.load_gather(codes_v, [d16])
                scale = jnp.minimum(1.0, ncv * nfv * 1e8)
                ce = acc * scale
                acc_v[nbuckets, :] = acc_v[nbuckets, :] + ce
                for t in range(_NUM_TRIALS):
                    ps = (csv >> (2 * t)) & 3
                    pd = (cdv >> (2 * t)) & 3
                    m = ps == pd
                    bidx = ps + (4 * t)
                    plsc.addupdate_scatter(acc_v, [bidx, lanes], ce, mask=m)
                    plsc.addupdate_scatter(
                        acc_v, [bidx + (nbuckets + 1), lanes], ones16, mask=m
                    )
                return 0

            lax.fori_loop(0, n_groups, group_body, 0)

        launch(0, 0)
        launch(1, 1)

        def outer_body(k, _):
            c0 = 2 * k
            wait(c0, 0)
            compute(c0, 0)
            launch(c0 + 2, 0)
            wait(c0 + 1, 1)
            compute(c0 + 1, 1)

            @pl.when(c0 + 3 < n_chunks)
            def _():
                launch(c0 + 3, 1)

            return 0

        lax.fori_loop(0, (n_chunks - 1) // 2, outer_body, 0)
        wait(n_chunks - 1, 0)
        compute(n_chunks - 1, 0)
        pltpu.sync_copy(acc_v, out_hbm.at[wid])

    return sc_kernel


def kernel(x, edge_index, Wc, bc, We, be):
    n, d = x.shape
    e = edge_index.shape[1]
    info = plsc.get_sparse_core_info()
    nc_cores, ns = info.num_cores, info.num_subcores
    nw = nc_cores * ns
    assert e % nw == 0
    e_per = e // nw
    ch = 80
    assert e_per % ch == 0 and ch % 16 == 0

    dense = pl.pallas_call(
        _dense_body,
        out_shape=[
            jax.ShapeDtypeStruct((n, d), jnp.bfloat16),
            jax.ShapeDtypeStruct((n, d), jnp.bfloat16),
            jax.ShapeDtypeStruct((n, 1), jnp.float32),
            jax.ShapeDtypeStruct((n, 1), jnp.float32),
        ],
    )
    chat, fhat, ncn, nfn = dense(
        x,
        Wc,
        bc.reshape(1, d),
        We,
        be.reshape(1, d),
    )
    ncn = ncn.reshape(n)
    nfn = nfn.reshape(n)
    codes = jnp.asarray(_partition_codes(n))
    src = edge_index[0].astype(jnp.int32)
    dst = edge_index[1].astype(jnp.int32)

    sc = _make_sc_kernel(n, d, e_per, ch, nw, nc_cores)
    parts = sc(chat, fhat, ncn, nfn, codes, src, dst)

    final = pl.pallas_call(
        functools.partial(_final_body, e=e),
        out_shape=jax.ShapeDtypeStruct((1, 3), jnp.float32),
    )
    out = final(parts)
    phi = out[0, 0]
    ei_whole = out[0, 1]
    min_pei = out[0, 2]
    return phi, ei_whole, min_pei
